# Initial kernel scaffold; baseline (speedup 1.0000x reference)
import jax
import jax.numpy as jnp
from jax.experimental import pallas as pl


def kernel(x, edge_index, W, b):
    raise NotImplementedError("write your pallas kernel here")


# speedup vs baseline: 17.4567x; 17.4567x over previous
import functools

import jax
import jax.numpy as jnp
from jax import lax
from jax.experimental import pallas as pl
from jax.experimental.pallas import tpu as pltpu
from jax.experimental.pallas import tpu_sc as plsc

N = 10000
E = 320000
D = 128
SCALE = 1.8

NC = 2
NS = 16
NW = NC * NS
E_PER_TILE = E // NW
CHUNK = 80
N_CHUNKS = E_PER_TILE // CHUNK
ROWS_A = 632
ROWS_B = N - 15 * ROWS_A
N_DEG = 10240
DEG_PER_TILE = N_DEG // NS

@functools.lru_cache(maxsize=None)
def _sc_mesh():
    return plsc.VectorSubcoreMesh(
        core_axis_name="c", subcore_axis_name="s", num_cores=NC, num_subcores=NS
    )


def _deg_body(dst_hbm, out_hbm, idx_v, ones_v, zeros_v, deg_sh):
    c = lax.axis_index("c")
    s = lax.axis_index("s")

    @pl.loop(0, CHUNK, step=16)
    def _(i):
        ones_v[pl.ds(i, 16)] = jnp.full((16,), 1.0, jnp.float32)

    @pl.loop(0, DEG_PER_TILE, step=16)
    def _(i):
        zeros_v[pl.ds(i, 16)] = jnp.zeros((16,), jnp.float32)

    pltpu.sync_copy(zeros_v, deg_sh.at[pl.ds(s * DEG_PER_TILE, DEG_PER_TILE)])
    plsc.subcore_barrier()

    base = (c * NS + s) * E_PER_TILE

    @pl.loop(0, N_CHUNKS)
    def _(i):
        pltpu.sync_copy(dst_hbm.at[pl.ds(base + i * CHUNK, CHUNK)], idx_v)
        pltpu.sync_copy(ones_v, deg_sh.at[idx_v], add=True)

    plsc.subcore_barrier()
    pltpu.sync_copy(
        deg_sh.at[pl.ds(s * DEG_PER_TILE, DEG_PER_TILE)],
        out_hbm.at[c, pl.ds(s * DEG_PER_TILE, DEG_PER_TILE)],
    )


@functools.lru_cache(maxsize=None)
def _deg_kernel():
    return pl.kernel(
        _deg_body,
        out_type=jax.ShapeDtypeStruct((NC, N_DEG), jnp.float32),
        mesh=_sc_mesh(),
        scratch_types=[
            pltpu.VMEM((CHUNK,), jnp.int32),
            pltpu.VMEM((CHUNK,), jnp.float32),
            pltpu.VMEM((DEG_PER_TILE,), jnp.float32),
            pltpu.VMEM_SHARED((N_DEG,), jnp.float32),
        ],
    )


def _msg_body(zs_hbm, src_hbm, dst_hbm, out_hbm, sidx, didx, rows, acc_sh):
    c = lax.axis_index("c")
    s = lax.axis_index("s")

    r0 = s * ROWS_A

    @pl.when(s < NS - 1)
    def _():
        pltpu.sync_copy(zs_hbm.at[pl.ds(r0, ROWS_A)],
                        acc_sh.at[pl.ds(r0, ROWS_A)])

    @pl.when(s == NS - 1)
    def _():
        pltpu.sync_copy(zs_hbm.at[pl.ds(15 * ROWS_A, ROWS_B)],
                        acc_sh.at[pl.ds(15 * ROWS_A, ROWS_B)])

    plsc.subcore_barrier()

    base = (c * NS + s) * E_PER_TILE

    @pl.loop(0, N_CHUNKS)
    def _(i):
        off = base + i * CHUNK
        pltpu.sync_copy(src_hbm.at[pl.ds(off, CHUNK)], sidx)
        pltpu.sync_copy(dst_hbm.at[pl.ds(off, CHUNK)], didx)
        pltpu.sync_copy(zs_hbm.at[sidx], rows)
        pltpu.sync_copy(rows, acc_sh.at[didx], add=True)

    plsc.subcore_barrier()

    @pl.when(s < NS - 1)
    def _():
        pltpu.sync_copy(acc_sh.at[pl.ds(r0, ROWS_A)],
                        out_hbm.at[c, pl.ds(r0, ROWS_A)])

    @pl.when(s == NS - 1)
    def _():
        pltpu.sync_copy(acc_sh.at[pl.ds(15 * ROWS_A, ROWS_B)],
                        out_hbm.at[c, pl.ds(15 * ROWS_A, ROWS_B)])


@functools.lru_cache(maxsize=None)
def _msg_kernel():
    return pl.kernel(
        _msg_body,
        out_type=jax.ShapeDtypeStruct((NC, N, D), jnp.float32),
        mesh=_sc_mesh(),
        scratch_types=[
            pltpu.VMEM((CHUNK,), jnp.int32),
            pltpu.VMEM((CHUNK,), jnp.int32),
            pltpu.VMEM((CHUNK, D), jnp.float32),
            pltpu.VMEM_SHARED((N, D), jnp.float32),
        ],
    )


def _proj_body(x_ref, w_ref, b_ref, d0_ref, d1_ref, zs_ref, dinv_ref):
    z = lax.dot_general(
        x_ref[...], w_ref[...], (((1,), (1,)), ((), ())),
        preferred_element_type=jnp.float32,
        precision=lax.Precision.HIGHEST,
    ) + b_ref[...]
    nrm = jnp.sqrt(jnp.sum(z * z, axis=1, keepdims=True))
    deg = d0_ref[...] + d1_ref[...] + 1.0
    dinv = lax.rsqrt(deg)
    scale = (SCALE * dinv) / jnp.maximum(nrm, 1e-12)
    zs_ref[...] = z * scale
    dinv_ref[...] = dinv


def _proj(x, W, b2, d0, d1, bn=1000):
    nblk = N // bn
    return pl.pallas_call(
        _proj_body,
        grid=(nblk,),
        in_specs=[
            pl.BlockSpec((bn, D), lambda i: (i, 0)),
            pl.BlockSpec((D, D), lambda i: (0, 0)),
            pl.BlockSpec((1, D), lambda i: (0, 0)),
            pl.BlockSpec((bn, 1), lambda i: (i, 0)),
            pl.BlockSpec((bn, 1), lambda i: (i, 0)),
        ],
        out_specs=[
            pl.BlockSpec((bn, D), lambda i: (i, 0)),
            pl.BlockSpec((bn, 1), lambda i: (i, 0)),
        ],
        out_shape=[
            jax.ShapeDtypeStruct((N, D), jnp.float32),
            jax.ShapeDtypeStruct((N, 1), jnp.float32),
        ],
    )(x, W, b2, d0, d1)


def _comb_body(p_ref, zs_ref, dinv_ref, out_ref):
    acc = p_ref[0] + p_ref[1] - zs_ref[...]
    out_ref[...] = dinv_ref[...] * acc


def _combine(parts, zs, dinv, bn=1000):
    nblk = N // bn
    return pl.pallas_call(
        _comb_body,
        grid=(nblk,),
        in_specs=[
            pl.BlockSpec((NC, bn, D), lambda i: (0, i, 0)),
            pl.BlockSpec((bn, D), lambda i: (i, 0)),
            pl.BlockSpec((bn, 1), lambda i: (i, 0)),
        ],
        out_specs=pl.BlockSpec((bn, D), lambda i: (i, 0)),
        out_shape=jax.ShapeDtypeStruct((N, D), jnp.float32),
    )(parts, zs, dinv)


@jax.jit
def kernel(x, edge_index, W, b):
    src = edge_index[0].astype(jnp.int32)
    dst = edge_index[1].astype(jnp.int32)

    deg_p = _deg_kernel()(dst)
    d0 = deg_p[0, :N].reshape(N, 1)
    d1 = deg_p[1, :N].reshape(N, 1)

    zs, dinv = _proj(x, W, b.reshape(1, D), d0, d1)
    parts = _msg_kernel()(zs, src, dst)
    return _combine(parts, zs, dinv)


# speedup vs baseline: 37.9269x; 2.1726x over previous
import functools

import jax
import jax.numpy as jnp
from jax import lax
from jax.experimental import pallas as pl
from jax.experimental.pallas import tpu as pltpu
from jax.experimental.pallas import tpu_sc as plsc

N = 10000
E = 320000
D = 128
SCALE = 1.8

NC = 2
NS = 16
NW = NC * NS
CHUNK = 128
NCHT = 80
E_PAD = NW * NCHT * CHUNK
EP_ROWS = E_PAD // CHUNK
N_ZS = N + 16
ZS_BN = 2504
ROWS_A = 632
ROWS_B = N - 15 * ROWS_A
N_DEG = 10240
DEG_PER_TILE = N_DEG // NS


@functools.lru_cache(maxsize=None)
def _sc_mesh():
    return plsc.VectorSubcoreMesh(
        core_axis_name="c", subcore_axis_name="s", num_cores=NC, num_subcores=NS
    )


def _deg_body(dst_hbm, out_hbm, didx, ones_v, zeros_v, dsem, deg_sh):
    c = lax.axis_index("c")
    s = lax.axis_index("s")
    wid = c * NS + s

    @pl.loop(0, CHUNK, step=16)
    def _(i):
        ones_v[pl.ds(i, 16)] = jnp.full((16,), 1.0, jnp.float32)

    @pl.loop(0, DEG_PER_TILE, step=16)
    def _(i):
        zeros_v[pl.ds(i, 16)] = jnp.zeros((16,), jnp.float32)

    pltpu.sync_copy(zeros_v, deg_sh.at[pl.ds(s * DEG_PER_TILE, DEG_PER_TILE)])
    pltpu.sync_copy(dst_hbm.at[pl.ds(wid * NCHT, NCHT)], didx)
    plsc.subcore_barrier()

    @pl.loop(0, NCHT)
    def _(j):
        pltpu.async_copy(ones_v, deg_sh.at[didx.at[j]], dsem, add=True)

    @pl.loop(0, NCHT)
    def _(j):
        pltpu.make_async_copy(ones_v, deg_sh.at[didx.at[j]], dsem).wait()

    plsc.subcore_barrier()
    pltpu.sync_copy(
        deg_sh.at[pl.ds(s * DEG_PER_TILE, DEG_PER_TILE)],
        out_hbm.at[c, pl.ds(s * DEG_PER_TILE, DEG_PER_TILE)],
    )


@functools.lru_cache(maxsize=None)
def _deg_kernel():
    return pl.kernel(
        _deg_body,
        out_type=jax.ShapeDtypeStruct((NC, N_DEG), jnp.float32),
        mesh=_sc_mesh(),
        scratch_types=[
            pltpu.VMEM((NCHT, CHUNK), jnp.int32),
            pltpu.VMEM((CHUNK,), jnp.float32),
            pltpu.VMEM((DEG_PER_TILE,), jnp.float32),
            pltpu.SemaphoreType.DMA,
            pltpu.VMEM_SHARED((N_DEG,), jnp.float32),
        ],
    )


def _msg_body(zs_hbm, src_hbm, dst_hbm, out_hbm,
              sidx, didx, rows, gsem_a, gsem_b, ssem_a, ssem_b, isem,
              acc_sh):
    c = lax.axis_index("c")
    s = lax.axis_index("s")
    wid = c * NS + s
    r0 = s * ROWS_A
    row0 = wid * NCHT

    @pl.when(s < NS - 1)
    def _():
        pltpu.sync_copy(zs_hbm.at[pl.ds(r0, ROWS_A)],
                        acc_sh.at[pl.ds(r0, ROWS_A)])

    @pl.when(s == NS - 1)
    def _():
        pltpu.sync_copy(zs_hbm.at[pl.ds(15 * ROWS_A, ROWS_B)],
                        acc_sh.at[pl.ds(15 * ROWS_A, ROWS_B)])

    plsc.subcore_barrier()

    gsem = (gsem_a, gsem_b)
    rbuf = (rows.at[0], rows.at[1])
    ssem = (ssem_a, ssem_b)

    def idx_load(j, slot):
        pltpu.async_copy(src_hbm.at[j], sidx.at[slot], isem.at[slot])
        pltpu.async_copy(dst_hbm.at[j], didx.at[slot], isem.at[slot])

    def idx_wait(j, slot):
        pltpu.make_async_copy(src_hbm.at[j], sidx.at[slot],
                              isem.at[slot]).wait()
        pltpu.make_async_copy(dst_hbm.at[j], didx.at[slot],
                              isem.at[slot]).wait()

    pltpu.sync_copy(src_hbm.at[row0], sidx.at[0])
    pltpu.sync_copy(dst_hbm.at[row0], didx.at[0])
    pltpu.async_copy(zs_hbm.at[sidx.at[0]], rbuf[0], gsem[0])
    idx_load(row0 + 1, 1)
    idx_load(row0 + 2, 2)

    @pl.loop(0, NCHT, step=4)
    def _(i):
        for b in range(4):
            j = i + b
            cur = b % 2
            nxt = (b + 1) % 2
            pltpu.make_async_copy(zs_hbm.at[sidx.at[b]], rbuf[cur],
                                  gsem[cur]).wait()

            if b == 0:
                @pl.when(j > 0)
                def _():
                    pltpu.make_async_copy(rbuf[nxt],
                                          acc_sh.at[didx.at[(b + 3) % 4]],
                                          ssem[nxt]).wait()
            else:
                pltpu.make_async_copy(rbuf[nxt],
                                      acc_sh.at[didx.at[(b + 3) % 4]],
                                      ssem[nxt]).wait()

            @pl.when(j + 3 < NCHT)
            def _():
                idx_load(row0 + j + 3, (b + 3) % 4)

            @pl.when(j + 1 < NCHT)
            def _():
                idx_wait(row0 + j + 1, (b + 1) % 4)
                pltpu.async_copy(zs_hbm.at[sidx.at[(b + 1) % 4]], rbuf[nxt],
                                 gsem[nxt])

            pltpu.async_copy(rbuf[cur], acc_sh.at[didx.at[b]], ssem[cur],
                             add=True)

    pltpu.make_async_copy(rbuf[1], acc_sh.at[didx.at[3]],
                          ssem[1]).wait()
    plsc.subcore_barrier()

    @pl.when(s < NS - 1)
    def _():
        pltpu.sync_copy(acc_sh.at[pl.ds(r0, ROWS_A)],
                        out_hbm.at[c, pl.ds(r0, ROWS_A)])

    @pl.when(s == NS - 1)
    def _():
        pltpu.sync_copy(acc_sh.at[pl.ds(15 * ROWS_A, ROWS_B)],
                        out_hbm.at[c, pl.ds(15 * ROWS_A, ROWS_B)])


@functools.lru_cache(maxsize=None)
def _msg_kernel():
    return pl.kernel(
        _msg_body,
        out_type=jax.ShapeDtypeStruct((NC, N, D), jnp.float32),
        mesh=_sc_mesh(),
        scratch_types=[
            pltpu.VMEM((4, CHUNK), jnp.int32),
            pltpu.VMEM((4, CHUNK), jnp.int32),
            pltpu.VMEM((2, CHUNK, D), jnp.float32),
            pltpu.SemaphoreType.DMA,
            pltpu.SemaphoreType.DMA,
            pltpu.SemaphoreType.DMA,
            pltpu.SemaphoreType.DMA,
            pltpu.SemaphoreType.DMA((4,)),
            pltpu.VMEM_SHARED((N, D), jnp.float32),
        ],
    )


def _proj_body(x_ref, w_ref, b_ref, zn_ref):
    z = lax.dot_general(
        x_ref[...], w_ref[...], (((1,), (1,)), ((), ())),
        preferred_element_type=jnp.float32,
        precision=lax.Precision.HIGHEST,
    ) + b_ref[...]
    nrm = jnp.sqrt(jnp.sum(z * z, axis=1, keepdims=True))
    zn_ref[...] = z * (SCALE / jnp.maximum(nrm, 1e-12))


def _proj(x, W, b2, bn=1000):
    return pl.pallas_call(
        _proj_body,
        grid=(N // bn,),
        in_specs=[
            pl.BlockSpec((bn, D), lambda i: (i, 0)),
            pl.BlockSpec((D, D), lambda i: (0, 0)),
            pl.BlockSpec((1, D), lambda i: (0, 0)),
        ],
        out_specs=pl.BlockSpec((bn, D), lambda i: (i, 0)),
        out_shape=jax.ShapeDtypeStruct((N, D), jnp.float32),
    )(x, W, b2)


def _scale_body(zn_ref, d0_ref, d1_ref, zs_ref, dinv_ref):
    i = pl.program_id(0)
    rows = lax.broadcasted_iota(jnp.int32, (ZS_BN, 1), 0) + i * ZS_BN
    mask = rows < N
    deg = d0_ref[...] + d1_ref[...] + 1.0
    dinv = jnp.where(mask, lax.rsqrt(deg), 0.0)
    zs_ref[...] = jnp.where(mask, zn_ref[...] * dinv, 0.0)
    dinv_ref[...] = dinv


def _scale(zn, d0, d1):
    return pl.pallas_call(
        _scale_body,
        grid=(N_ZS // ZS_BN,),
        in_specs=[
            pl.BlockSpec((ZS_BN, D), lambda i: (i, 0)),
            pl.BlockSpec((ZS_BN, 1), lambda i: (i, 0)),
            pl.BlockSpec((ZS_BN, 1), lambda i: (i, 0)),
        ],
        out_specs=[
            pl.BlockSpec((ZS_BN, D), lambda i: (i, 0)),
            pl.BlockSpec((ZS_BN, 1), lambda i: (i, 0)),
        ],
        out_shape=[
            jax.ShapeDtypeStruct((N_ZS, D), jnp.float32),
            jax.ShapeDtypeStruct((N_ZS, 1), jnp.float32),
        ],
    )(zn, d0, d1)


def _comb_body(p_ref, zs_ref, dinv_ref, out_ref):
    out_ref[...] = dinv_ref[...] * (p_ref[0] + p_ref[1] - zs_ref[...])


def _combine(parts, zs, dinv, bn=1000):
    return pl.pallas_call(
        _comb_body,
        grid=(N // bn,),
        in_specs=[
            pl.BlockSpec((NC, bn, D), lambda i: (0, i, 0)),
            pl.BlockSpec((bn, D), lambda i: (i, 0)),
            pl.BlockSpec((bn, 1), lambda i: (i, 0)),
        ],
        out_specs=pl.BlockSpec((bn, D), lambda i: (i, 0)),
        out_shape=jax.ShapeDtypeStruct((N, D), jnp.float32),
    )(parts, zs, dinv)


@jax.jit
def kernel(x, edge_index, W, b):
    src = edge_index[0].astype(jnp.int32)
    dst = edge_index[1].astype(jnp.int32)

    pad = E_PAD - E
    ar = jnp.arange(pad, dtype=jnp.int32)
    src2 = jnp.concatenate([src, N + (ar % 16)]).reshape(EP_ROWS, CHUNK)
    dst2m = jnp.concatenate([dst, (ar * 131) % N]).reshape(EP_ROWS, CHUNK)
    dst2d = jnp.concatenate([dst, N + (ar % 16)]).reshape(EP_ROWS, CHUNK)

    deg_p = _deg_kernel()(dst2d)
    zn = _proj(x, W, b.reshape(1, D))

    d0 = deg_p[0, :N].reshape(N, 1)
    d1 = deg_p[1, :N].reshape(N, 1)
    zs, dinv = _scale(zn, d0, d1)

    parts = _msg_kernel()(zs, src2, dst2m)
    return _combine(parts, zs, dinv)
